# Initial kernel scaffold; baseline (speedup 1.0000x reference)
import jax
import jax.numpy as jnp
from jax.experimental import pallas as pl


def kernel(x):
    raise NotImplementedError("write your pallas kernel here")


# speedup vs baseline: 37.5019x; 37.5019x over previous
import functools

import jax
import jax.numpy as jnp
from jax import lax
from jax.experimental import pallas as pl
from jax.experimental.pallas import tpu as pltpu
from jax.experimental.pallas import tpu_sc as plsc

_CONF = 0.1
_IOU = 0.7
_K = 2048
_NB = 16
_B = 128
_MAXDET = 300
_SX = 1920.0 / 1280.0
_SY = 1080.0 / 1024.0
_OUT_PAD = 320


def _transpose_col(row, eye):
    return lax.dot_general(
        eye, row, (((1,), (1,)), ((), ())),
        preferred_element_type=jnp.float32,
        precision=lax.Precision.HIGHEST,
    )


def _iou_block(rx1, ry1, rx2, ry2, rar, cx1, cy1, cx2, cy2, car):
    xx1 = jnp.maximum(rx1, cx1)
    yy1 = jnp.maximum(ry1, cy1)
    xx2 = jnp.minimum(rx2, cx2)
    yy2 = jnp.minimum(ry2, cy2)
    iw = jnp.maximum(xx2 - xx1, 0.0)
    ih = jnp.maximum(yy2 - yy1, 0.0)
    inter = iw * ih
    union = rar + car - inter
    return jnp.where(union > 0.0, inter / union, 0.0)


def _nms_body(cx_ref, cy_ref, w_ref, h_ref,
              keeps,
              x1s, y1s, x2s, y2s, ars):
    b = pl.program_id(0)

    @pl.when(b == 0)
    def _init():
        cx = cx_ref[...]
        cy = cy_ref[...]
        w2 = w_ref[...] * 0.5
        h2 = h_ref[...] * 0.5
        x1s[...] = cx - w2
        y1s[...] = cy - h2
        x2s[...] = cx + w2
        y2s[...] = cy + h2
        ars[...] = (x2s[...] - x1s[...]) * (y2s[...] - y1s[...])
        keeps[...] = jnp.zeros((_NB, _B), jnp.float32)

    ii = lax.broadcasted_iota(jnp.int32, (_B, _B), 0)
    jj = lax.broadcasted_iota(jnp.int32, (_B, _B), 1)
    eye = (ii == jj).astype(jnp.float32)

    bx1 = x1s[pl.ds(b, 1), :]
    by1 = y1s[pl.ds(b, 1), :]
    bx2 = x2s[pl.ds(b, 1), :]
    by2 = y2s[pl.ds(b, 1), :]
    bar = ars[pl.ds(b, 1), :]
    rx1 = _transpose_col(bx1, eye)
    ry1 = _transpose_col(by1, eye)
    rx2 = _transpose_col(bx2, eye)
    ry2 = _transpose_col(by2, eye)
    rar = _transpose_col(bar, eye)

    def cross(c, sup):
        cx1 = x1s[pl.ds(c, 1), :]
        cy1 = y1s[pl.ds(c, 1), :]
        cx2 = x2s[pl.ds(c, 1), :]
        cy2 = y2s[pl.ds(c, 1), :]
        car = ars[pl.ds(c, 1), :]
        kc = keeps[pl.ds(c, 1), :]
        iou = _iou_block(rx1, ry1, rx2, ry2, rar, cx1, cy1, cx2, cy2, car)
        contrib = jnp.where(iou > _IOU, kc, 0.0)
        return jnp.maximum(sup, jnp.max(contrib, axis=1, keepdims=True))

    sup = lax.fori_loop(0, b, cross, jnp.zeros((_B, 1), jnp.float32))

    iou_d = _iou_block(rx1, ry1, rx2, ry2, rar, bx1, by1, bx2, by2, bar)
    m = (iou_d > _IOU).astype(jnp.float32)
    lane = lax.broadcasted_iota(jnp.int32, (1, _B), 1)
    kb = jnp.ones((1, _B), jnp.float32)
    for i in range(_B):
        row = lax.slice(m, (i, 0), (i + 1, _B))
        lt = (lane < i).astype(jnp.float32)
        s_in = jnp.max(row * kb * lt, axis=1, keepdims=True)
        s_tot = jnp.maximum(s_in, lax.slice(sup, (i, 0), (i + 1, 1)))
        oh = (lane == i).astype(jnp.float32)
        kb = kb - oh * kb * s_tot
    keeps[pl.ds(b, 1), :] = kb


def _nms(cx, cy, w, h):
    spec_full = pl.BlockSpec((_NB, _B), lambda b: (0, 0))
    return pl.pallas_call(
        _nms_body,
        grid=(_NB,),
        in_specs=[spec_full] * 4,
        out_specs=pl.BlockSpec((_NB, _B), lambda b: (0, 0)),
        out_shape=jax.ShapeDtypeStruct((_NB, _B), jnp.float32),
        scratch_shapes=[pltpu.VMEM((_NB, _B), jnp.float32)] * 5,
    )(cx, cy, w, h)


def _select_body(keep_hbm, cx_hbm, cy_hbm, w_hbm, h_hbm, sc_hbm, out_hbm,
                 keep_v, cx_v, cy_v, w_v, h_v, sc_v, out_v):
    wid = lax.axis_index("s") * 2 + lax.axis_index("c")

    @pl.when(wid == 0)
    def _():
        pltpu.sync_copy(keep_hbm, keep_v)
        pltpu.sync_copy(cx_hbm, cx_v)
        pltpu.sync_copy(cy_hbm, cy_v)
        pltpu.sync_copy(w_hbm, w_v)
        pltpu.sync_copy(h_hbm, h_v)
        pltpu.sync_copy(sc_hbm, sc_v)

        zeros = jnp.zeros((16,), jnp.float32)

        def zbody(i, carry):
            out_v[pl.ds(i * 16, 16)] = zeros
            return carry

        lax.fori_loop(0, _OUT_PAD * 5 // 16, zbody, 0)

        def body(j, count):
            base = j * 16
            k = keep_v[pl.ds(base, 16)]
            s = sc_v[pl.ds(base, 16)]
            msk = (k > 0.5) & (s >= _CONF)
            mi = msk.astype(jnp.int32)
            pos = count + plsc.cumsum(mi) - 1
            ok = msk & (pos < _MAXDET)
            flat = pos * 5
            cxv = cx_v[pl.ds(base, 16)] * _SX + 1.0
            cyv = cy_v[pl.ds(base, 16)] * _SY + 1.0
            wv = w_v[pl.ds(base, 16)] * _SX
            hv = h_v[pl.ds(base, 16)] * _SY
            plsc.store_scatter(out_v, [flat], cxv, mask=ok)
            plsc.store_scatter(out_v, [flat + 1], cyv, mask=ok)
            plsc.store_scatter(out_v, [flat + 2], wv, mask=ok)
            plsc.store_scatter(out_v, [flat + 3], hv, mask=ok)
            plsc.store_scatter(out_v, [flat + 4], s, mask=ok)
            return count + jnp.sum(mi)

        lax.fori_loop(0, _K // 16, body, jnp.int32(0))
        pltpu.sync_copy(out_v, out_hbm)


@functools.cache
def _build_select():
    mesh = plsc.VectorSubcoreMesh(core_axis_name="c", subcore_axis_name="s")
    return pl.kernel(
        _select_body,
        mesh=mesh,
        out_type=jax.ShapeDtypeStruct((_OUT_PAD * 5,), jnp.float32),
        scratch_types=[
            pltpu.VMEM((_K,), jnp.float32),
            pltpu.VMEM((_K,), jnp.float32),
            pltpu.VMEM((_K,), jnp.float32),
            pltpu.VMEM((_K,), jnp.float32),
            pltpu.VMEM((_K,), jnp.float32),
            pltpu.VMEM((_K,), jnp.float32),
            pltpu.VMEM((_OUT_PAD * 5,), jnp.float32),
        ],
        compiler_params=pltpu.CompilerParams(needs_layout_passes=False),
    )


def kernel(x):
    scores_all = x[4]
    masked = jnp.where(scores_all >= _CONF, scores_all, -jnp.inf)
    _, top_idx = lax.top_k(masked, _K)
    cand = x[:, top_idx]
    cx = cand[0].reshape(_NB, _B)
    cy = cand[1].reshape(_NB, _B)
    w = cand[2].reshape(_NB, _B)
    h = cand[3].reshape(_NB, _B)
    keep = _nms(cx, cy, w, h)
    out_flat = _build_select()(keep.reshape(_K), cand[0], cand[1], cand[2],
                               cand[3], cand[4])
    return out_flat.reshape(_OUT_PAD, 5)[:_MAXDET]


# speedup vs baseline: 104.9400x; 2.7983x over previous
import functools

import jax
import jax.numpy as jnp
from jax import lax
from jax.experimental import pallas as pl
from jax.experimental.pallas import tpu as pltpu
from jax.experimental.pallas import tpu_sc as plsc

_CONF = 0.1
_IOU = 0.7
_K = 2048
_NB = 16
_B = 128
_MAXDET = 300
_SX = 1920.0 / 1280.0
_SY = 1080.0 / 1024.0
_OUT_PAD = 320


def _transpose_col(row, eye):
    return lax.dot_general(
        eye, row, (((1,), (1,)), ((), ())),
        preferred_element_type=jnp.float32,
        precision=lax.Precision.HIGHEST,
    )


def _transpose_row(col, eye):
    return lax.dot_general(
        col, eye, (((0,), (0,)), ((), ())),
        preferred_element_type=jnp.float32,
        precision=lax.Precision.HIGHEST,
    )


def _iou_block(rx1, ry1, rx2, ry2, rar, cx1, cy1, cx2, cy2, car):
    xx1 = jnp.maximum(rx1, cx1)
    yy1 = jnp.maximum(ry1, cy1)
    xx2 = jnp.minimum(rx2, cx2)
    yy2 = jnp.minimum(ry2, cy2)
    iw = jnp.maximum(xx2 - xx1, 0.0)
    ih = jnp.maximum(yy2 - yy1, 0.0)
    inter = iw * ih
    union = rar + car - inter
    return jnp.where(union > 0.0, inter / union, 0.0)


def _nms_body(cx_ref, cy_ref, w_ref, h_ref,
              keeps,
              x1s, y1s, x2s, y2s, ars):
    b = pl.program_id(0)

    @pl.when(b == 0)
    def _init():
        cx = cx_ref[...]
        cy = cy_ref[...]
        w2 = w_ref[...] * 0.5
        h2 = h_ref[...] * 0.5
        x1s[...] = cx - w2
        y1s[...] = cy - h2
        x2s[...] = cx + w2
        y2s[...] = cy + h2
        ars[...] = (x2s[...] - x1s[...]) * (y2s[...] - y1s[...])
        keeps[...] = jnp.zeros((_NB, _B), jnp.float32)

    ii = lax.broadcasted_iota(jnp.int32, (_B, _B), 0)
    jj = lax.broadcasted_iota(jnp.int32, (_B, _B), 1)
    eye = (ii == jj).astype(jnp.float32)

    bx1 = x1s[pl.ds(b, 1), :]
    by1 = y1s[pl.ds(b, 1), :]
    bx2 = x2s[pl.ds(b, 1), :]
    by2 = y2s[pl.ds(b, 1), :]
    bar = ars[pl.ds(b, 1), :]
    rx1 = _transpose_col(bx1, eye)
    ry1 = _transpose_col(by1, eye)
    rx2 = _transpose_col(bx2, eye)
    ry2 = _transpose_col(by2, eye)
    rar = _transpose_col(bar, eye)

    def cross(c, sup):
        cx1 = x1s[pl.ds(c, 1), :]
        cy1 = y1s[pl.ds(c, 1), :]
        cx2 = x2s[pl.ds(c, 1), :]
        cy2 = y2s[pl.ds(c, 1), :]
        car = ars[pl.ds(c, 1), :]
        kc = keeps[pl.ds(c, 1), :]
        iou = _iou_block(rx1, ry1, rx2, ry2, rar, cx1, cy1, cx2, cy2, car)
        contrib = jnp.where(iou > _IOU, kc, 0.0)
        return jnp.maximum(sup, jnp.max(contrib, axis=1, keepdims=True))

    sup = lax.fori_loop(0, b, cross, jnp.zeros((_B, 1), jnp.float32))

    iou_d = _iou_block(rx1, ry1, rx2, ry2, rar, bx1, by1, bx2, by2, bar)
    strict_lt = lax.broadcasted_iota(jnp.int32, (_B, _B), 1) < \
        lax.broadcasted_iota(jnp.int32, (_B, _B), 0)
    m = jnp.where((iou_d > _IOU) & strict_lt, 1.0, 0.0)

    def not_converged(carry):
        kb, changed = carry
        return changed

    def step(carry):
        kb, _ = carry
        s = jnp.maximum(jnp.max(m * kb, axis=1, keepdims=True), sup)
        kb_new = _transpose_row(1.0 - jnp.minimum(s, 1.0), eye)
        return kb_new, jnp.any(kb_new != kb)

    kb0 = jnp.ones((1, _B), jnp.float32)
    kb, _ = lax.while_loop(not_converged, step, (kb0, jnp.bool_(True)))
    keeps[pl.ds(b, 1), :] = kb


def _nms(cx, cy, w, h):
    spec_full = pl.BlockSpec((_NB, _B), lambda b: (0, 0))
    return pl.pallas_call(
        _nms_body,
        grid=(_NB,),
        in_specs=[spec_full] * 4,
        out_specs=pl.BlockSpec((_NB, _B), lambda b: (0, 0)),
        out_shape=jax.ShapeDtypeStruct((_NB, _B), jnp.float32),
        scratch_shapes=[pltpu.VMEM((_NB, _B), jnp.float32)] * 5,
    )(cx, cy, w, h)


def _select_body(keep_hbm, cx_hbm, cy_hbm, w_hbm, h_hbm, sc_hbm, out_hbm,
                 keep_v, cx_v, cy_v, w_v, h_v, sc_v, out_v):
    wid = lax.axis_index("s") * 2 + lax.axis_index("c")

    @pl.when(wid == 0)
    def _():
        pltpu.sync_copy(keep_hbm, keep_v)
        pltpu.sync_copy(cx_hbm, cx_v)
        pltpu.sync_copy(cy_hbm, cy_v)
        pltpu.sync_copy(w_hbm, w_v)
        pltpu.sync_copy(h_hbm, h_v)
        pltpu.sync_copy(sc_hbm, sc_v)

        zeros = jnp.zeros((16,), jnp.float32)

        def zbody(i, carry):
            out_v[pl.ds(i * 16, 16)] = zeros
            return carry

        lax.fori_loop(0, _OUT_PAD * 5 // 16, zbody, 0)

        def body(j, count):
            base = j * 16
            k = keep_v[pl.ds(base, 16)]
            s = sc_v[pl.ds(base, 16)]
            msk = (k > 0.5) & (s >= _CONF)
            mi = msk.astype(jnp.int32)
            pos = count + plsc.cumsum(mi) - 1
            ok = msk & (pos < _MAXDET)
            flat = pos * 5
            cxv = cx_v[pl.ds(base, 16)] * _SX + 1.0
            cyv = cy_v[pl.ds(base, 16)] * _SY + 1.0
            wv = w_v[pl.ds(base, 16)] * _SX
            hv = h_v[pl.ds(base, 16)] * _SY
            plsc.store_scatter(out_v, [flat], cxv, mask=ok)
            plsc.store_scatter(out_v, [flat + 1], cyv, mask=ok)
            plsc.store_scatter(out_v, [flat + 2], wv, mask=ok)
            plsc.store_scatter(out_v, [flat + 3], hv, mask=ok)
            plsc.store_scatter(out_v, [flat + 4], s, mask=ok)
            return count + jnp.sum(mi)

        lax.fori_loop(0, _K // 16, body, jnp.int32(0))
        pltpu.sync_copy(out_v, out_hbm)


@functools.cache
def _build_select():
    mesh = plsc.VectorSubcoreMesh(core_axis_name="c", subcore_axis_name="s")
    return pl.kernel(
        _select_body,
        mesh=mesh,
        out_type=jax.ShapeDtypeStruct((_OUT_PAD * 5,), jnp.float32),
        scratch_types=[
            pltpu.VMEM((_K,), jnp.float32),
            pltpu.VMEM((_K,), jnp.float32),
            pltpu.VMEM((_K,), jnp.float32),
            pltpu.VMEM((_K,), jnp.float32),
            pltpu.VMEM((_K,), jnp.float32),
            pltpu.VMEM((_K,), jnp.float32),
            pltpu.VMEM((_OUT_PAD * 5,), jnp.float32),
        ],
        compiler_params=pltpu.CompilerParams(needs_layout_passes=False),
    )


def kernel(x):
    scores_all = x[4]
    masked = jnp.where(scores_all >= _CONF, scores_all, -jnp.inf)
    _, top_idx = lax.top_k(masked, _K)
    cand = x[:, top_idx]
    cx = cand[0].reshape(_NB, _B)
    cy = cand[1].reshape(_NB, _B)
    w = cand[2].reshape(_NB, _B)
    h = cand[3].reshape(_NB, _B)
    keep = _nms(cx, cy, w, h)
    out_flat = _build_select()(keep.reshape(_K), cand[0], cand[1], cand[2],
                               cand[3], cand[4])
    return out_flat.reshape(_OUT_PAD, 5)[:_MAXDET]


# speedup vs baseline: 149.0221x; 1.4201x over previous
import functools

import jax
import jax.numpy as jnp
from jax import lax
from jax.experimental import pallas as pl
from jax.experimental.pallas import tpu as pltpu
from jax.experimental.pallas import tpu_sc as plsc

_CONF = 0.1
_IOU = 0.7
_K = 2048
_NB = 16
_B = 128
_MAXDET = 300
_SX = 1920.0 / 1280.0
_SY = 1080.0 / 1024.0
_OUT_PAD = 320


def _transpose_col(row, eye):
    return lax.dot_general(
        eye, row, (((1,), (1,)), ((), ())),
        preferred_element_type=jnp.float32,
        precision=lax.Precision.HIGHEST,
    )


def _transpose_row(col, eye):
    return lax.dot_general(
        col, eye, (((0,), (0,)), ((), ())),
        preferred_element_type=jnp.float32,
        precision=lax.Precision.HIGHEST,
    )


def _iou_block(rx1, ry1, rx2, ry2, rar, cx1, cy1, cx2, cy2, car):
    xx1 = jnp.maximum(rx1, cx1)
    yy1 = jnp.maximum(ry1, cy1)
    xx2 = jnp.minimum(rx2, cx2)
    yy2 = jnp.minimum(ry2, cy2)
    iw = jnp.maximum(xx2 - xx1, 0.0)
    ih = jnp.maximum(yy2 - yy1, 0.0)
    inter = iw * ih
    union = rar + car - inter
    return jnp.where(union > 0.0, inter / union, 0.0)


def _nms_body(cx_ref, cy_ref, w_ref, h_ref, sc_ref,
              keeps,
              x1s, y1s, x2s, y2s, ars, cnt_ref):
    b = pl.program_id(0)

    @pl.when(b == 0)
    def _init():
        cx = cx_ref[...]
        cy = cy_ref[...]
        w2 = w_ref[...] * 0.5
        h2 = h_ref[...] * 0.5
        x1s[...] = cx - w2
        y1s[...] = cy - h2
        x2s[...] = cx + w2
        y2s[...] = cy + h2
        ars[...] = (x2s[...] - x1s[...]) * (y2s[...] - y1s[...])
        keeps[...] = jnp.zeros((_NB, _B), jnp.float32)
        cnt_ref[0] = 0

    @pl.when(cnt_ref[0] < _MAXDET)
    def _block():
        ii = lax.broadcasted_iota(jnp.int32, (_B, _B), 0)
        jj = lax.broadcasted_iota(jnp.int32, (_B, _B), 1)
        eye = (ii == jj).astype(jnp.float32)

        bx1 = x1s[pl.ds(b, 1), :]
        by1 = y1s[pl.ds(b, 1), :]
        bx2 = x2s[pl.ds(b, 1), :]
        by2 = y2s[pl.ds(b, 1), :]
        bar = ars[pl.ds(b, 1), :]
        rx1 = _transpose_col(bx1, eye)
        ry1 = _transpose_col(by1, eye)
        rx2 = _transpose_col(bx2, eye)
        ry2 = _transpose_col(by2, eye)
        rar = _transpose_col(bar, eye)

        def cross(c, sup):
            cx1 = x1s[pl.ds(c, 1), :]
            cy1 = y1s[pl.ds(c, 1), :]
            cx2 = x2s[pl.ds(c, 1), :]
            cy2 = y2s[pl.ds(c, 1), :]
            car = ars[pl.ds(c, 1), :]
            kc = keeps[pl.ds(c, 1), :]
            iou = _iou_block(rx1, ry1, rx2, ry2, rar, cx1, cy1, cx2, cy2, car)
            contrib = jnp.where(iou > _IOU, kc, 0.0)
            return jnp.maximum(sup, jnp.max(contrib, axis=1, keepdims=True))

        sup = lax.fori_loop(0, b, cross, jnp.zeros((_B, 1), jnp.float32))

        iou_d = _iou_block(rx1, ry1, rx2, ry2, rar, bx1, by1, bx2, by2, bar)
        strict_lt = jj < ii
        m = jnp.where((iou_d > _IOU) & strict_lt, 1.0, 0.0)

        def not_converged(carry):
            kb, changed = carry
            return changed

        def step(carry):
            kb, _ = carry
            s = jnp.maximum(jnp.max(m * kb, axis=1, keepdims=True), sup)
            kb_new = _transpose_row(1.0 - jnp.minimum(s, 1.0), eye)
            return kb_new, jnp.any(kb_new != kb)

        kb0 = jnp.ones((1, _B), jnp.float32)
        kb, _ = lax.while_loop(not_converged, step, (kb0, jnp.bool_(True)))
        keeps[pl.ds(b, 1), :] = kb

        valid = jnp.where(sc_ref[pl.ds(b, 1), :] >= _CONF, kb, 0.0)
        cnt_ref[0] = cnt_ref[0] + jnp.sum(valid).astype(jnp.int32)


def _nms(cx, cy, w, h, sc):
    spec_full = pl.BlockSpec((_NB, _B), lambda b: (0, 0))
    return pl.pallas_call(
        _nms_body,
        grid=(_NB,),
        in_specs=[spec_full] * 5,
        out_specs=pl.BlockSpec((_NB, _B), lambda b: (0, 0)),
        out_shape=jax.ShapeDtypeStruct((_NB, _B), jnp.float32),
        scratch_shapes=[pltpu.VMEM((_NB, _B), jnp.float32)] * 5
        + [pltpu.SMEM((1,), jnp.int32)],
    )(cx, cy, w, h, sc)


def _select_body(keep_hbm, cx_hbm, cy_hbm, w_hbm, h_hbm, sc_hbm, out_hbm,
                 keep_v, cx_v, cy_v, w_v, h_v, sc_v, out_v):
    wid = lax.axis_index("s") * 2 + lax.axis_index("c")

    @pl.when(wid == 0)
    def _():
        pltpu.sync_copy(keep_hbm, keep_v)
        pltpu.sync_copy(cx_hbm, cx_v)
        pltpu.sync_copy(cy_hbm, cy_v)
        pltpu.sync_copy(w_hbm, w_v)
        pltpu.sync_copy(h_hbm, h_v)
        pltpu.sync_copy(sc_hbm, sc_v)

        zeros = jnp.zeros((16,), jnp.float32)

        def zbody(i, carry):
            out_v[pl.ds(i * 16, 16)] = zeros
            return carry

        lax.fori_loop(0, _OUT_PAD * 5 // 16, zbody, 0)

        def body(j, count):
            base = j * 16
            k = keep_v[pl.ds(base, 16)]
            s = sc_v[pl.ds(base, 16)]
            msk = (k > 0.5) & (s >= _CONF)
            mi = msk.astype(jnp.int32)
            pos = count + plsc.cumsum(mi) - 1
            ok = msk & (pos < _MAXDET)
            flat = pos * 5
            cxv = cx_v[pl.ds(base, 16)] * _SX + 1.0
            cyv = cy_v[pl.ds(base, 16)] * _SY + 1.0
            wv = w_v[pl.ds(base, 16)] * _SX
            hv = h_v[pl.ds(base, 16)] * _SY
            plsc.store_scatter(out_v, [flat], cxv, mask=ok)
            plsc.store_scatter(out_v, [flat + 1], cyv, mask=ok)
            plsc.store_scatter(out_v, [flat + 2], wv, mask=ok)
            plsc.store_scatter(out_v, [flat + 3], hv, mask=ok)
            plsc.store_scatter(out_v, [flat + 4], s, mask=ok)
            return count + jnp.sum(mi)

        lax.fori_loop(0, _K // 16, body, jnp.int32(0))
        pltpu.sync_copy(out_v, out_hbm)


@functools.cache
def _build_select():
    mesh = plsc.VectorSubcoreMesh(core_axis_name="c", subcore_axis_name="s")
    return pl.kernel(
        _select_body,
        mesh=mesh,
        out_type=jax.ShapeDtypeStruct((_OUT_PAD * 5,), jnp.float32),
        scratch_types=[
            pltpu.VMEM((_K,), jnp.float32),
            pltpu.VMEM((_K,), jnp.float32),
            pltpu.VMEM((_K,), jnp.float32),
            pltpu.VMEM((_K,), jnp.float32),
            pltpu.VMEM((_K,), jnp.float32),
            pltpu.VMEM((_K,), jnp.float32),
            pltpu.VMEM((_OUT_PAD * 5,), jnp.float32),
        ],
        compiler_params=pltpu.CompilerParams(needs_layout_passes=False),
    )


def kernel(x):
    scores_all = x[4]
    masked = jnp.where(scores_all >= _CONF, scores_all, -jnp.inf)
    _, top_idx = lax.top_k(masked, _K)
    cand = x[:, top_idx]
    cx = cand[0].reshape(_NB, _B)
    cy = cand[1].reshape(_NB, _B)
    w = cand[2].reshape(_NB, _B)
    h = cand[3].reshape(_NB, _B)
    sc = cand[4].reshape(_NB, _B)
    keep = _nms(cx, cy, w, h, sc)
    out_flat = _build_select()(keep.reshape(_K), cand[0], cand[1], cand[2],
                               cand[3], cand[4])
    return out_flat.reshape(_OUT_PAD, 5)[:_MAXDET]


# speedup vs baseline: 168.1505x; 1.1284x over previous
import functools

import jax
import jax.numpy as jnp
from jax import lax
from jax.experimental import pallas as pl
from jax.experimental.pallas import tpu as pltpu
from jax.experimental.pallas import tpu_sc as plsc

_CONF = 0.1
_IOU = 0.7
_K = 2048
_NB = 16
_B = 128
_MAXDET = 300
_SX = 1920.0 / 1280.0
_SY = 1080.0 / 1024.0
_OUT_PAD = 320


def _transpose_col(row, eye):
    return lax.dot_general(
        eye, row, (((1,), (1,)), ((), ())),
        preferred_element_type=jnp.float32,
        precision=lax.Precision.HIGHEST,
    )


def _transpose_row(col, eye):
    return lax.dot_general(
        col, eye, (((0,), (0,)), ((), ())),
        preferred_element_type=jnp.float32,
        precision=lax.Precision.HIGHEST,
    )


def _iou_block(rx1, ry1, rx2, ry2, rar, cx1, cy1, cx2, cy2, car):
    xx1 = jnp.maximum(rx1, cx1)
    yy1 = jnp.maximum(ry1, cy1)
    xx2 = jnp.minimum(rx2, cx2)
    yy2 = jnp.minimum(ry2, cy2)
    iw = jnp.maximum(xx2 - xx1, 0.0)
    ih = jnp.maximum(yy2 - yy1, 0.0)
    inter = iw * ih
    union = rar + car - inter
    return jnp.where(union > 0.0, inter / union, 0.0)


def _nms_body(cx_ref, cy_ref, w_ref, h_ref, sc_ref,
              keeps,
              x1s, y1s, x2s, y2s, ars, cnt_ref):
    b = pl.program_id(0)

    @pl.when(b == 0)
    def _init():
        cx = cx_ref[...]
        cy = cy_ref[...]
        w2 = w_ref[...] * 0.5
        h2 = h_ref[...] * 0.5
        x1s[...] = cx - w2
        y1s[...] = cy - h2
        x2s[...] = cx + w2
        y2s[...] = cy + h2
        ars[...] = (x2s[...] - x1s[...]) * (y2s[...] - y1s[...])
        keeps[...] = jnp.zeros((_NB, _B), jnp.float32)
        cnt_ref[0] = 0

    @pl.when(cnt_ref[0] < _MAXDET)
    def _block():
        ii = lax.broadcasted_iota(jnp.int32, (_B, _B), 0)
        jj = lax.broadcasted_iota(jnp.int32, (_B, _B), 1)
        eye = (ii == jj).astype(jnp.float32)

        bx1 = x1s[pl.ds(b, 1), :]
        by1 = y1s[pl.ds(b, 1), :]
        bx2 = x2s[pl.ds(b, 1), :]
        by2 = y2s[pl.ds(b, 1), :]
        bar = ars[pl.ds(b, 1), :]
        rx1 = _transpose_col(bx1, eye)
        ry1 = _transpose_col(by1, eye)
        rx2 = _transpose_col(bx2, eye)
        ry2 = _transpose_col(by2, eye)
        rar = _transpose_col(bar, eye)

        def cross(c, sup):
            cx1 = x1s[pl.ds(c, 1), :]
            cy1 = y1s[pl.ds(c, 1), :]
            cx2 = x2s[pl.ds(c, 1), :]
            cy2 = y2s[pl.ds(c, 1), :]
            car = ars[pl.ds(c, 1), :]
            kc = keeps[pl.ds(c, 1), :]
            iou = _iou_block(rx1, ry1, rx2, ry2, rar, cx1, cy1, cx2, cy2, car)
            contrib = jnp.where(iou > _IOU, kc, 0.0)
            return jnp.maximum(sup, jnp.max(contrib, axis=1, keepdims=True))

        sup = lax.fori_loop(0, b, cross, jnp.zeros((_B, 1), jnp.float32))

        iou_d = _iou_block(rx1, ry1, rx2, ry2, rar, bx1, by1, bx2, by2, bar)
        strict_lt = jj < ii
        m = jnp.where((iou_d > _IOU) & strict_lt, 1.0, 0.0)

        def not_converged(carry):
            kb, changed = carry
            return changed

        def step(carry):
            kb, _ = carry
            s = jnp.maximum(jnp.max(m * kb, axis=1, keepdims=True), sup)
            kb_new = _transpose_row(1.0 - jnp.minimum(s, 1.0), eye)
            return kb_new, jnp.any(kb_new != kb)

        kb0 = jnp.ones((1, _B), jnp.float32)
        kb, _ = lax.while_loop(not_converged, step, (kb0, jnp.bool_(True)))
        keeps[pl.ds(b, 1), :] = kb

        valid = jnp.where(sc_ref[pl.ds(b, 1), :] >= _CONF, kb, 0.0)
        cnt_ref[0] = cnt_ref[0] + jnp.sum(valid).astype(jnp.int32)


def _nms(cx, cy, w, h, sc):
    spec_full = pl.BlockSpec((_NB, _B), lambda b: (0, 0))
    return pl.pallas_call(
        _nms_body,
        grid=(_NB,),
        in_specs=[spec_full] * 5,
        out_specs=pl.BlockSpec((_NB, _B), lambda b: (0, 0)),
        out_shape=jax.ShapeDtypeStruct((_NB, _B), jnp.float32),
        scratch_shapes=[pltpu.VMEM((_NB, _B), jnp.float32)] * 5
        + [pltpu.SMEM((1,), jnp.int32)],
    )(cx, cy, w, h, sc)


def _select_body(keep_hbm, cx_hbm, cy_hbm, w_hbm, h_hbm, sc_hbm, out_hbm,
                 keep_v, cx_v, cy_v, w_v, h_v, sc_v, out_v):
    wid = lax.axis_index("s") * 2 + lax.axis_index("c")

    @pl.when(wid == 0)
    def _():
        pltpu.sync_copy(keep_hbm, keep_v)
        pltpu.sync_copy(cx_hbm, cx_v)
        pltpu.sync_copy(cy_hbm, cy_v)
        pltpu.sync_copy(w_hbm, w_v)
        pltpu.sync_copy(h_hbm, h_v)
        pltpu.sync_copy(sc_hbm, sc_v)

        zeros = jnp.zeros((16,), jnp.float32)

        def zbody(i, carry):
            out_v[pl.ds(i * 16, 16)] = zeros
            return carry

        lax.fori_loop(0, _OUT_PAD * 5 // 16, zbody, 0)

        def body(j, count):
            base = j * 16
            k = keep_v[pl.ds(base, 16)]
            s = sc_v[pl.ds(base, 16)]
            msk = (k > 0.5) & (s >= _CONF)
            mi = msk.astype(jnp.int32)
            pos = count + plsc.cumsum(mi) - 1
            ok = msk & (pos < _MAXDET)
            flat = pos * 5
            cxv = cx_v[pl.ds(base, 16)] * _SX + 1.0
            cyv = cy_v[pl.ds(base, 16)] * _SY + 1.0
            wv = w_v[pl.ds(base, 16)] * _SX
            hv = h_v[pl.ds(base, 16)] * _SY
            plsc.store_scatter(out_v, [flat], cxv, mask=ok)
            plsc.store_scatter(out_v, [flat + 1], cyv, mask=ok)
            plsc.store_scatter(out_v, [flat + 2], wv, mask=ok)
            plsc.store_scatter(out_v, [flat + 3], hv, mask=ok)
            plsc.store_scatter(out_v, [flat + 4], s, mask=ok)
            return count + jnp.sum(mi)

        lax.fori_loop(0, _K // 16, body, jnp.int32(0))
        pltpu.sync_copy(out_v, out_hbm)


_GPW = _K // 32


def _gather_body(x0, x1, x2, x3, x4, idx_hbm,
                 o0, o1, o2, o3, o4, idx_v, val_v, sem):
    wid = lax.axis_index("s") * 2 + lax.axis_index("c")
    base = wid * _GPW
    pltpu.sync_copy(idx_hbm.at[pl.ds(base, _GPW)], idx_v)
    for src, dst in ((x0, o0), (x1, o1), (x2, o2), (x3, o3), (x4, o4)):
        pltpu.async_copy(src.at[idx_v], val_v, sem).wait()
        pltpu.sync_copy(val_v, dst.at[pl.ds(base, _GPW)])


@functools.cache
def _build_gather():
    mesh = plsc.VectorSubcoreMesh(core_axis_name="c", subcore_axis_name="s")
    return pl.kernel(
        _gather_body,
        mesh=mesh,
        out_type=[jax.ShapeDtypeStruct((_K,), jnp.float32)] * 5,
        scratch_types=[
            pltpu.VMEM((_GPW,), jnp.int32),
            pltpu.VMEM((_GPW,), jnp.float32),
            pltpu.SemaphoreType.DMA,
        ],
        compiler_params=pltpu.CompilerParams(needs_layout_passes=False),
    )


@functools.cache
def _build_select():
    mesh = plsc.VectorSubcoreMesh(core_axis_name="c", subcore_axis_name="s")
    return pl.kernel(
        _select_body,
        mesh=mesh,
        out_type=jax.ShapeDtypeStruct((_OUT_PAD * 5,), jnp.float32),
        scratch_types=[
            pltpu.VMEM((_K,), jnp.float32),
            pltpu.VMEM((_K,), jnp.float32),
            pltpu.VMEM((_K,), jnp.float32),
            pltpu.VMEM((_K,), jnp.float32),
            pltpu.VMEM((_K,), jnp.float32),
            pltpu.VMEM((_K,), jnp.float32),
            pltpu.VMEM((_OUT_PAD * 5,), jnp.float32),
        ],
        compiler_params=pltpu.CompilerParams(needs_layout_passes=False),
    )


def kernel(x):
    scores_all = x[4]
    masked = jnp.where(scores_all >= _CONF, scores_all, -jnp.inf)
    _, top_idx = lax.top_k(masked, _K)
    gcx, gcy, gw, gh, gsc = _build_gather()(x[0], x[1], x[2], x[3], x[4],
                                            top_idx)
    keep = _nms(gcx.reshape(_NB, _B), gcy.reshape(_NB, _B),
                gw.reshape(_NB, _B), gh.reshape(_NB, _B),
                gsc.reshape(_NB, _B))
    out_flat = _build_select()(keep.reshape(_K), gcx, gcy, gw, gh, gsc)
    return out_flat.reshape(_OUT_PAD, 5)[:_MAXDET]


# speedup vs baseline: 181.6614x; 1.0804x over previous
import functools

import jax
import jax.numpy as jnp
from jax import lax
from jax.experimental import pallas as pl
from jax.experimental.pallas import tpu as pltpu
from jax.experimental.pallas import tpu_sc as plsc

_CONF = 0.1
_IOU = 0.7
_K = 2048
_NB = 16
_B = 128
_MAXDET = 300
_SX = 1920.0 / 1280.0
_SY = 1080.0 / 1024.0


def _transpose_col(row, eye):
    return lax.dot_general(
        eye, row, (((1,), (1,)), ((), ())),
        preferred_element_type=jnp.float32,
        precision=lax.Precision.HIGHEST,
    )


def _transpose_row(col, eye):
    return lax.dot_general(
        col, eye, (((0,), (0,)), ((), ())),
        preferred_element_type=jnp.float32,
        precision=lax.Precision.HIGHEST,
    )


def _iou_block(rx1, ry1, rx2, ry2, rar, cx1, cy1, cx2, cy2, car):
    xx1 = jnp.maximum(rx1, cx1)
    yy1 = jnp.maximum(ry1, cy1)
    xx2 = jnp.minimum(rx2, cx2)
    yy2 = jnp.minimum(ry2, cy2)
    iw = jnp.maximum(xx2 - xx1, 0.0)
    ih = jnp.maximum(yy2 - yy1, 0.0)
    inter = iw * ih
    union = rar + car - inter
    return jnp.where(union > 0.0, inter / union, 0.0)


_OROW = 304


def _nms_body(cx_ref, cy_ref, w_ref, h_ref, sc_ref,
              out_ref,
              keeps, x1s, y1s, x2s, y2s, ars, cnt_ref):
    b = pl.program_id(0)

    @pl.when(b == 0)
    def _init():
        cx = cx_ref[...]
        cy = cy_ref[...]
        w2 = w_ref[...] * 0.5
        h2 = h_ref[...] * 0.5
        x1s[...] = cx - w2
        y1s[...] = cy - h2
        x2s[...] = cx + w2
        y2s[...] = cy + h2
        ars[...] = (x2s[...] - x1s[...]) * (y2s[...] - y1s[...])
        keeps[...] = jnp.zeros((_NB, _B), jnp.float32)
        cnt_ref[0] = 0

    @pl.when(cnt_ref[0] < _MAXDET)
    def _block():
        ii = lax.broadcasted_iota(jnp.int32, (_B, _B), 0)
        jj = lax.broadcasted_iota(jnp.int32, (_B, _B), 1)
        eye = (ii == jj).astype(jnp.float32)

        bx1 = x1s[pl.ds(b, 1), :]
        by1 = y1s[pl.ds(b, 1), :]
        bx2 = x2s[pl.ds(b, 1), :]
        by2 = y2s[pl.ds(b, 1), :]
        bar = ars[pl.ds(b, 1), :]
        rx1 = _transpose_col(bx1, eye)
        ry1 = _transpose_col(by1, eye)
        rx2 = _transpose_col(bx2, eye)
        ry2 = _transpose_col(by2, eye)
        rar = _transpose_col(bar, eye)

        def cross(c, sup):
            cx1 = x1s[pl.ds(c, 1), :]
            cy1 = y1s[pl.ds(c, 1), :]
            cx2 = x2s[pl.ds(c, 1), :]
            cy2 = y2s[pl.ds(c, 1), :]
            car = ars[pl.ds(c, 1), :]
            kc = keeps[pl.ds(c, 1), :]
            iou = _iou_block(rx1, ry1, rx2, ry2, rar, cx1, cy1, cx2, cy2, car)
            contrib = jnp.where(iou > _IOU, kc, 0.0)
            return jnp.maximum(sup, jnp.max(contrib, axis=1, keepdims=True))

        sup = lax.fori_loop(0, b, cross, jnp.zeros((_B, 1), jnp.float32))

        iou_d = _iou_block(rx1, ry1, rx2, ry2, rar, bx1, by1, bx2, by2, bar)
        strict_lt = jj < ii
        m = jnp.where((iou_d > _IOU) & strict_lt, 1.0, 0.0)

        def not_converged(carry):
            kb, changed = carry
            return changed

        def step(carry):
            kb, _ = carry
            s = jnp.maximum(jnp.max(m * kb, axis=1, keepdims=True), sup)
            kb_new = _transpose_row(1.0 - jnp.minimum(s, 1.0), eye)
            return kb_new, jnp.any(kb_new != kb)

        kb0 = jnp.ones((1, _B), jnp.float32)
        kb, _ = lax.while_loop(not_converged, step, (kb0, jnp.bool_(True)))
        keeps[pl.ds(b, 1), :] = kb

        valid = jnp.where(sc_ref[pl.ds(b, 1), :] >= _CONF, kb, 0.0)
        cnt_ref[0] = cnt_ref[0] + jnp.sum(valid).astype(jnp.int32)

    @pl.when(b == _NB - 1)
    def _emit():
        ii = lax.broadcasted_iota(jnp.int32, (_B, _B), 0)
        jj = lax.broadcasted_iota(jnp.int32, (_B, _B), 1)
        eyeb = (ii == jj).astype(jnp.float32)
        ustrict = (ii < jj).astype(jnp.float32)
        hi = lax.Precision.HIGHEST

        valid = jnp.where((sc_ref[...] >= _CONF) & (keeps[...] > 0.5),
                          1.0, 0.0)
        pos_in = lax.dot_general(valid, ustrict, (((1,), (0,)), ((), ())),
                                 preferred_element_type=jnp.float32,
                                 precision=hi)
        rowsum = pos_in[:, _B - 1:_B] + valid[:, _B - 1:_B]
        r16 = lax.broadcasted_iota(jnp.int32, (_NB, _NB), 0)
        s16 = lax.broadcasted_iota(jnp.int32, (_NB, _NB), 1)
        lstrict = (s16 < r16).astype(jnp.float32)
        row_off = lax.dot_general(lstrict, rowsum, (((1,), (0,)), ((), ())),
                                  preferred_element_type=jnp.float32,
                                  precision=hi)
        p = row_off + pos_in
        pfin = jnp.where((valid > 0.5) & (p < _MAXDET), p, 1e9)

        def t16(a):
            return lax.dot_general(eyeb, a, (((1,), (1,)), ((), ())),
                                   preferred_element_type=jnp.float32,
                                   precision=hi)

        pT = t16(pfin)
        vT = [t16(cx_ref[...] * _SX + 1.0), t16(cy_ref[...] * _SY + 1.0),
              t16(w_ref[...] * _SX), t16(h_ref[...] * _SY), t16(sc_ref[...])]
        kio = lax.broadcasted_iota(jnp.int32, (1, _OROW), 1).astype(jnp.float32)
        zpad = jnp.zeros((_B, 3), jnp.float32)
        acc = jnp.zeros((_OROW, 8), jnp.float32)
        for r in range(_NB):
            pr = pT[:, r:r + 1]
            oh = (pr == kio).astype(jnp.float32)
            vals = jnp.concatenate([v[:, r:r + 1] for v in vT] + [zpad],
                                   axis=1)
            acc = acc + lax.dot_general(oh, vals, (((0,), (0,)), ((), ())),
                                        preferred_element_type=jnp.float32,
                                        precision=hi)
        out_ref[...] = acc


def _nms(cx, cy, w, h, sc):
    spec_full = pl.BlockSpec((_NB, _B), lambda b: (0, 0))
    return pl.pallas_call(
        _nms_body,
        grid=(_NB,),
        in_specs=[spec_full] * 5,
        out_specs=pl.BlockSpec((_OROW, 8), lambda b: (0, 0)),
        out_shape=jax.ShapeDtypeStruct((_OROW, 8), jnp.float32),
        scratch_shapes=[pltpu.VMEM((_NB, _B), jnp.float32)] * 6
        + [pltpu.SMEM((1,), jnp.int32)],
    )(cx, cy, w, h, sc)


_GPW = _K // 32


def _gather_body(x0, x1, x2, x3, x4, idx_hbm,
                 o0, o1, o2, o3, o4, idx_v, val_v, sem):
    wid = lax.axis_index("s") * 2 + lax.axis_index("c")
    base = wid * _GPW
    pltpu.sync_copy(idx_hbm.at[pl.ds(base, _GPW)], idx_v)
    for src, dst in ((x0, o0), (x1, o1), (x2, o2), (x3, o3), (x4, o4)):
        pltpu.async_copy(src.at[idx_v], val_v, sem).wait()
        pltpu.sync_copy(val_v, dst.at[pl.ds(base, _GPW)])


@functools.cache
def _build_gather():
    mesh = plsc.VectorSubcoreMesh(core_axis_name="c", subcore_axis_name="s")
    return pl.kernel(
        _gather_body,
        mesh=mesh,
        out_type=[jax.ShapeDtypeStruct((_K,), jnp.float32)] * 5,
        scratch_types=[
            pltpu.VMEM((_GPW,), jnp.int32),
            pltpu.VMEM((_GPW,), jnp.float32),
            pltpu.SemaphoreType.DMA,
        ],
        compiler_params=pltpu.CompilerParams(needs_layout_passes=False),
    )


def kernel(x):
    scores_all = x[4]
    masked = jnp.where(scores_all >= _CONF, scores_all, -jnp.inf)
    _, top_idx = lax.top_k(masked, _K)
    gcx, gcy, gw, gh, gsc = _build_gather()(x[0], x[1], x[2], x[3], x[4],
                                            top_idx)
    out = _nms(gcx.reshape(_NB, _B), gcy.reshape(_NB, _B),
               gw.reshape(_NB, _B), gh.reshape(_NB, _B),
               gsc.reshape(_NB, _B))
    return out[:_MAXDET, :5]


# speedup vs baseline: 198.2134x; 1.0911x over previous
import functools

import jax
import jax.numpy as jnp
from jax import lax
from jax.experimental import pallas as pl
from jax.experimental.pallas import tpu as pltpu
from jax.experimental.pallas import tpu_sc as plsc

_CONF = 0.1
_IOU = 0.7
_K = 2048
_NB = 16
_B = 128
_MAXDET = 300
_SX = 1920.0 / 1280.0
_SY = 1080.0 / 1024.0


def _transpose_col(row, eye):
    return lax.dot_general(
        eye, row, (((1,), (1,)), ((), ())),
        preferred_element_type=jnp.float32,
        precision=lax.Precision.HIGHEST,
    )


def _transpose_row(col, eye):
    return lax.dot_general(
        col, eye, (((0,), (0,)), ((), ())),
        preferred_element_type=jnp.float32,
        precision=lax.Precision.HIGHEST,
    )


def _iou_block(rx1, ry1, rx2, ry2, rar, cx1, cy1, cx2, cy2, car):
    xx1 = jnp.maximum(rx1, cx1)
    yy1 = jnp.maximum(ry1, cy1)
    xx2 = jnp.minimum(rx2, cx2)
    yy2 = jnp.minimum(ry2, cy2)
    iw = jnp.maximum(xx2 - xx1, 0.0)
    ih = jnp.maximum(yy2 - yy1, 0.0)
    inter = iw * ih
    union = rar + car - inter
    return jnp.where(union > 0.0, inter / union, 0.0)


_OROW = 304


def _nms_body(cx_ref, cy_ref, w_ref, h_ref, sc_ref,
              out_ref,
              keeps, x1s, y1s, x2s, y2s, ars, cnt_ref):
    b = pl.program_id(0)

    @pl.when(b == 0)
    def _init():
        cx = cx_ref[...]
        cy = cy_ref[...]
        w2 = w_ref[...] * 0.5
        h2 = h_ref[...] * 0.5
        x1s[...] = cx - w2
        y1s[...] = cy - h2
        x2s[...] = cx + w2
        y2s[...] = cy + h2
        ars[...] = (x2s[...] - x1s[...]) * (y2s[...] - y1s[...])
        keeps[...] = jnp.zeros((_NB, _B), jnp.float32)
        cnt_ref[0] = 0

    @pl.when(cnt_ref[0] < _MAXDET)
    def _block():
        ii = lax.broadcasted_iota(jnp.int32, (_B, _B), 0)
        jj = lax.broadcasted_iota(jnp.int32, (_B, _B), 1)
        eye = (ii == jj).astype(jnp.float32)

        bx1 = x1s[pl.ds(b, 1), :]
        by1 = y1s[pl.ds(b, 1), :]
        bx2 = x2s[pl.ds(b, 1), :]
        by2 = y2s[pl.ds(b, 1), :]
        bar = ars[pl.ds(b, 1), :]
        rx1 = _transpose_col(bx1, eye)
        ry1 = _transpose_col(by1, eye)
        rx2 = _transpose_col(bx2, eye)
        ry2 = _transpose_col(by2, eye)
        rar = _transpose_col(bar, eye)

        def cross(c, sup):
            cx1 = x1s[pl.ds(c, 1), :]
            cy1 = y1s[pl.ds(c, 1), :]
            cx2 = x2s[pl.ds(c, 1), :]
            cy2 = y2s[pl.ds(c, 1), :]
            car = ars[pl.ds(c, 1), :]
            kc = keeps[pl.ds(c, 1), :]
            iou = _iou_block(rx1, ry1, rx2, ry2, rar, cx1, cy1, cx2, cy2, car)
            contrib = jnp.where(iou > _IOU, kc, 0.0)
            return jnp.maximum(sup, jnp.max(contrib, axis=1, keepdims=True))

        sup = lax.fori_loop(0, b, cross, jnp.zeros((_B, 1), jnp.float32))

        iou_d = _iou_block(rx1, ry1, rx2, ry2, rar, bx1, by1, bx2, by2, bar)
        strict_lt = jj < ii
        m = jnp.where((iou_d > _IOU) & strict_lt, 1.0, 0.0)

        def not_converged(carry):
            kb, changed = carry
            return changed

        def step(carry):
            kb, _ = carry
            s = jnp.maximum(jnp.max(m * kb, axis=1, keepdims=True), sup)
            kb_new = _transpose_row(1.0 - jnp.minimum(s, 1.0), eye)
            return kb_new, jnp.any(kb_new != kb)

        kb0 = jnp.ones((1, _B), jnp.float32)
        kb, _ = lax.while_loop(not_converged, step, (kb0, jnp.bool_(True)))
        keeps[pl.ds(b, 1), :] = kb

        valid = jnp.where(sc_ref[pl.ds(b, 1), :] >= _CONF, kb, 0.0)
        cnt_ref[0] = cnt_ref[0] + jnp.sum(valid).astype(jnp.int32)

    @pl.when(b == _NB - 1)
    def _emit():
        ii = lax.broadcasted_iota(jnp.int32, (_B, _B), 0)
        jj = lax.broadcasted_iota(jnp.int32, (_B, _B), 1)
        eyeb = (ii == jj).astype(jnp.float32)
        ustrict = (ii < jj).astype(jnp.float32)
        hi = lax.Precision.HIGHEST

        valid = jnp.where((sc_ref[...] >= _CONF) & (keeps[...] > 0.5),
                          1.0, 0.0)
        pos_in = lax.dot_general(valid, ustrict, (((1,), (0,)), ((), ())),
                                 preferred_element_type=jnp.float32,
                                 precision=hi)
        rowsum = pos_in[:, _B - 1:_B] + valid[:, _B - 1:_B]
        r16 = lax.broadcasted_iota(jnp.int32, (_NB, _NB), 0)
        s16 = lax.broadcasted_iota(jnp.int32, (_NB, _NB), 1)
        lstrict = (s16 < r16).astype(jnp.float32)
        row_off = lax.dot_general(lstrict, rowsum, (((1,), (0,)), ((), ())),
                                  preferred_element_type=jnp.float32,
                                  precision=hi)
        p = row_off + pos_in
        pfin = jnp.where((valid > 0.5) & (p < _MAXDET), p, 1e9)

        def t16(a):
            return lax.dot_general(eyeb, a, (((1,), (1,)), ((), ())),
                                   preferred_element_type=jnp.float32,
                                   precision=hi)

        pT = t16(pfin)
        vT = [t16(cx_ref[...] * _SX + 1.0), t16(cy_ref[...] * _SY + 1.0),
              t16(w_ref[...] * _SX), t16(h_ref[...] * _SY), t16(sc_ref[...])]
        kio = lax.broadcasted_iota(jnp.int32, (1, _OROW), 1).astype(jnp.float32)
        zpad = jnp.zeros((_B, 3), jnp.float32)
        acc = jnp.zeros((_OROW, 8), jnp.float32)
        for r in range(_NB):
            pr = pT[:, r:r + 1]
            oh = (pr == kio).astype(jnp.float32)
            vals = jnp.concatenate([v[:, r:r + 1] for v in vT] + [zpad],
                                   axis=1)
            acc = acc + lax.dot_general(oh, vals, (((0,), (0,)), ((), ())),
                                        preferred_element_type=jnp.float32,
                                        precision=hi)
        out_ref[...] = acc


def _nms(cx, cy, w, h, sc):
    spec_full = pl.BlockSpec((_NB, _B), lambda b: (0, 0))
    return pl.pallas_call(
        _nms_body,
        grid=(_NB,),
        in_specs=[spec_full] * 5,
        out_specs=pl.BlockSpec((_OROW, 8), lambda b: (0, 0)),
        out_shape=jax.ShapeDtypeStruct((_OROW, 8), jnp.float32),
        scratch_shapes=[pltpu.VMEM((_NB, _B), jnp.float32)] * 6
        + [pltpu.SMEM((1,), jnp.int32)],
    )(cx, cy, w, h, sc)


_GPW = _K // 32


_N = 20000


def _gather_body(xflat, idx_hbm, o0, o1, o2, o3, o4,
                 idx_v, i0, i1, i2, i3, i4, v0, v1, v2, v3, v4, sem):
    wid = lax.axis_index("s") * 2 + lax.axis_index("c")
    base = wid * _GPW
    pltpu.sync_copy(idx_hbm.at[pl.ds(base, _GPW)], idx_v)
    idxr = (i0, i1, i2, i3, i4)
    vals = (v0, v1, v2, v3, v4)
    outs = (o0, o1, o2, o3, o4)
    for r in range(5):
        for c in range(_GPW // 16):
            idxr[r][pl.ds(c * 16, 16)] = idx_v[pl.ds(c * 16, 16)] + r * _N
    copies = [pltpu.async_copy(xflat.at[idxr[r]], vals[r], sem)
              for r in range(5)]
    for r in range(5):
        copies[r].wait()
        pltpu.sync_copy(vals[r], outs[r].at[pl.ds(base, _GPW)])


@functools.cache
def _build_gather():
    mesh = plsc.VectorSubcoreMesh(core_axis_name="c", subcore_axis_name="s")
    return pl.kernel(
        _gather_body,
        mesh=mesh,
        out_type=[jax.ShapeDtypeStruct((_K,), jnp.float32)] * 5,
        scratch_types=[pltpu.VMEM((_GPW,), jnp.int32)] * 6
        + [pltpu.VMEM((_GPW,), jnp.float32)] * 5
        + [pltpu.SemaphoreType.DMA],
        compiler_params=pltpu.CompilerParams(needs_layout_passes=False),
    )


def kernel(x):
    scores_all = x[4]
    masked = jnp.where(scores_all >= _CONF, scores_all, -jnp.inf)
    _, top_idx = lax.top_k(masked, _K)
    gcx, gcy, gw, gh, gsc = _build_gather()(x.reshape(5 * _N), top_idx)
    out = _nms(gcx.reshape(_NB, _B), gcy.reshape(_NB, _B),
               gw.reshape(_NB, _B), gh.reshape(_NB, _B),
               gsc.reshape(_NB, _B))
    return out[:_MAXDET, :5]
